# baseline (speedup 1.0000x reference)
import jax
import jax.numpy as jnp
from jax.experimental import pallas as pl
from jax.experimental.pallas import tpu as pltpu

D_MODEL = 1024
D_FF = 4096
N_BLOCK = 32
D1 = 128
D_LOWRANK = 64
TBC = 512
TBM = 256
NPAIR = 8192 // (2 * TBM)


def _ctrl_body(x_ref, m1_ref, m2_ref, mb_ref, am_ref):
    xt = x_ref[...]
    t1 = jnp.dot(xt, m1_ref[...], preferred_element_type=jnp.float32)
    lg = jnp.dot(t1, m2_ref[...], preferred_element_type=jnp.float32)
    lg = lg + mb_ref[...]
    m = lg[:, 0:D1]
    am = jnp.zeros((TBC, D1), dtype=jnp.int32)
    for y in range(1, N_BLOCK):
        ly = lg[:, y * D1:(y + 1) * D1]
        gt = ly > m
        am = jnp.where(gt, y, am)
        m = jnp.where(gt, ly, m)
    am_ref[...] = am


def _select(mid, am):
    zeros = jnp.zeros((TBM, D1), dtype=jnp.float32)
    pieces = []
    for y in range(N_BLOCK):
        my = mid[:, y * D1:(y + 1) * D1]
        pieces.append(jnp.where(am == y, jnp.maximum(my, 0.0), zeros))
    return jnp.concatenate(pieces, axis=1)


def _main_body(x_ref, am_ref, w1_ref, w2_ref, b2_ref, out_ref, bufa, bufb):
    b2v = b2_ref[...]
    outa = jnp.dot(bufa[...], w2_ref[...], preferred_element_type=jnp.float32)
    out_ref[0:TBM, :] = outa + b2v
    outb = jnp.dot(bufb[...], w2_ref[...], preferred_element_type=jnp.float32)
    out_ref[TBM:2 * TBM, :] = outb + b2v
    xt = x_ref[...]
    am = am_ref[...]
    w1 = w1_ref[...]
    mida = jnp.dot(xt[0:TBM, :], w1, preferred_element_type=jnp.float32)
    bufa[...] = _select(mida, am[0:TBM, :])
    midb = jnp.dot(xt[TBM:2 * TBM, :], w1, preferred_element_type=jnp.float32)
    bufb[...] = _select(midb, am[TBM:2 * TBM, :])


@jax.jit
def kernel(x, m1, m2, mb, w1, w2, b2):
    B, S, _ = x.shape
    T = B * S
    xf = x.reshape(T, D_MODEL)
    m2f = m2.transpose(0, 2, 1).reshape(D_LOWRANK, D_FF)
    mbf = mb.transpose(1, 0).reshape(1, D_FF)
    w1f = w1.transpose(0, 2, 1).reshape(D_MODEL, D_FF)
    w2f = w2.reshape(D_FF, D_MODEL)
    b2f = b2.reshape(1, D_MODEL)

    am = pl.pallas_call(
        _ctrl_body,
        grid=(T // TBC,),
        in_specs=[
            pl.BlockSpec((TBC, D_MODEL), lambda i: (i, 0)),
            pl.BlockSpec((D_MODEL, D_LOWRANK), lambda i: (0, 0)),
            pl.BlockSpec((D_LOWRANK, D_FF), lambda i: (0, 0)),
            pl.BlockSpec((1, D_FF), lambda i: (0, 0)),
        ],
        out_specs=pl.BlockSpec((TBC, D1), lambda i: (i, 0)),
        out_shape=jax.ShapeDtypeStruct((T, D1), jnp.int32),
    )(xf, m1, m2f, mbf)
    out = pl.pallas_call(
        _main_body,
        grid=(NPAIR + 1,),
        in_specs=[
            pl.BlockSpec((2 * TBM, D_MODEL),
                         lambda j: (jnp.minimum(j, NPAIR - 1), 0)),
            pl.BlockSpec((2 * TBM, D1),
                         lambda j: (jnp.minimum(j, NPAIR - 1), 0)),
            pl.BlockSpec((D_MODEL, D_FF), lambda j: (0, 0)),
            pl.BlockSpec((D_FF, D_MODEL), lambda j: (0, 0)),
            pl.BlockSpec((1, D_MODEL), lambda j: (0, 0)),
        ],
        out_specs=pl.BlockSpec((2 * TBM, D_MODEL),
                               lambda j: (jnp.maximum(j - 1, 0), 0)),
        out_shape=jax.ShapeDtypeStruct((T, D_MODEL), jnp.float32),
        scratch_shapes=[pltpu.VMEM((TBM, D_FF), jnp.float32),
                        pltpu.VMEM((TBM, D_FF), jnp.float32)],
    )(xf, am, w1f, w2f, b2f)
    return out.reshape(B, S, D_MODEL)
